# baseline (speedup 1.0000x reference)
import functools

import jax
import jax.numpy as jnp
from jax import lax
from jax.experimental import pallas as pl
from jax.experimental.pallas import tpu as pltpu
from jax.experimental.pallas import tpu_sc as plsc

_R = 0.86
_L = 16
_NW = 32
_CH = 1024
_NB = 2048
_HPAD = 2056
_NCAND = 1024
_LN2 = 0.6931471805599453


def _ilog_poly(s):
    bits = plsc.bitcast(s, jnp.int32)
    ex = lax.shift_right_logical(bits, 23) - 127
    mb = lax.bitwise_or(lax.bitwise_and(bits, 0x7FFFFF), 0x3F800000)
    m = plsc.bitcast(mb, jnp.float32)
    z = (m - 1.0) / (m + 1.0)
    z2 = z * z
    p = 1.0 / 9.0 + z2 * 0.0
    p = 1.0 / 7.0 + z2 * p
    p = 1.0 / 5.0 + z2 * p
    p = 1.0 / 3.0 + z2 * p
    p = 1.0 + z2 * p
    return ex.astype(jnp.float32) * _LN2 + 2.0 * z * p


def _search(hist, a0, target, nbins, lane):
    lo = jnp.zeros((_L,), jnp.int32)
    hi = jnp.full((_L,), nbins - 1, jnp.int32)
    steps = nbins.bit_length() - 1

    def step(_, carry):
        lo, hi = carry
        mid = lax.shift_right_logical(lo + hi + 1, 1)
        sv = plsc.load_gather(hist, [mid, lane])
        c = (a0 + sv) >= target
        return jnp.where(c, mid, lo), jnp.where(c, hi, mid - 1)

    lo, hi = lax.fori_loop(0, steps, step, (lo, hi))
    anext = a0 + plsc.load_gather(hist, [lo + 1, lane])
    return lo, anext


def _zero_hist(hist, nrows):
    def z(j, _):
        hist[j] = jnp.zeros((_L,), jnp.float32)
        return 0

    lax.fori_loop(0, nrows, z, 0)


def _suffix_sum(hist, nbins):
    def sfx(j, acc):
        jr = nbins - 1 - j
        acc = acc + hist[jr]
        hist[jr] = acc
        return acc

    lax.fori_loop(0, nbins, sfx, jnp.zeros((_L,), jnp.float32))


def _sc_body(x_hbm, o_hbm, buf, obuf, hist, cand_e, cand_k):
    wid = lax.axis_index("s") * 2 + lax.axis_index("c")
    lane = lax.iota(jnp.int32, _L)
    B, V, C = 64, 8192, 32
    nchunks = V // _CH

    def run_unit(t, _):
        unit = t * _NW + wid
        b = lax.shift_right_logical(unit, 1)
        h = lax.bitwise_and(unit, 1)
        col0 = h * _L

        _zero_hist(hist, _HPAD)

        def scan1_chunk(ci, s_acc):
            pltpu.sync_copy(
                x_hbm.at[b, pl.ds(ci * _CH, _CH), pl.ds(col0, _L)], buf)

            def row(i, s_acc):
                e = jnp.exp(buf[i])
                k = plsc.bitcast(e, jnp.int32)
                b1 = lax.shift_right_logical(k, 20)
                plsc.addupdate_scatter(hist, [b1, lane], e)
                return s_acc + e

            return lax.fori_loop(0, _CH, row, s_acc)

        s = lax.fori_loop(0, nchunks, scan1_chunk, jnp.zeros((_L,), jnp.float32))
        target = _R * s

        _suffix_sum(hist, _NB)
        b1s, a1 = _search(hist, jnp.zeros((_L,), jnp.float32), target, _NB, lane)

        def scan2_chunk(ci, cnt):
            pltpu.sync_copy(
                x_hbm.at[b, pl.ds(ci * _CH, _CH), pl.ds(col0, _L)], buf)

            def row(i, cnt):
                e = jnp.exp(buf[i])
                k = plsc.bitcast(e, jnp.int32)
                m = (lax.shift_right_logical(k, 20) == b1s) & (cnt < _NCAND)
                plsc.store_scatter(cand_e, [cnt, lane], e, mask=m)
                plsc.store_scatter(cand_k, [cnt, lane], k, mask=m)
                return cnt + jnp.where(m, 1, 0)

            return lax.fori_loop(0, _CH, row, cnt)

        cnt = lax.fori_loop(0, nchunks, scan2_chunk, jnp.zeros((_L,), jnp.int32))

        nmax = jnp.max(cnt)

        _zero_hist(hist, _HPAD)

        def r2(j, _):
            valid = (lane * 0 + j) < cnt
            k = cand_k[j]
            b2 = lax.bitwise_and(lax.shift_right_logical(k, 9), 0x7FF)
            plsc.addupdate_scatter(hist, [b2, lane], cand_e[j], mask=valid)
            return 0

        lax.fori_loop(0, nmax, r2, 0)
        _suffix_sum(hist, _NB)
        b2s, a2 = _search(hist, a1, target, _NB, lane)

        _zero_hist(hist, 520)

        def r3(j, _):
            k = cand_k[j]
            valid = ((lane * 0 + j) < cnt) & (
                lax.bitwise_and(lax.shift_right_logical(k, 9), 0x7FF) == b2s)
            b3 = lax.bitwise_and(k, 0x1FF)
            plsc.addupdate_scatter(hist, [b3, lane], cand_e[j], mask=valid)
            return 0

        lax.fori_loop(0, nmax, r3, 0)
        _suffix_sum(hist, 512)
        b3s, _ = _search(hist, a2, target, 512, lane)

        tau_k = lax.bitwise_or(
            lax.bitwise_or(lax.shift_left(b1s, 20), lax.shift_left(b2s, 9)),
            b3s)
        tau = plsc.bitcast(tau_k, jnp.float32)
        logz = _ilog_poly(s)

        def scan3_chunk(ci, _):
            pltpu.sync_copy(
                x_hbm.at[b, pl.ds(ci * _CH, _CH), pl.ds(col0, _L)], buf)

            def row(i, _):
                v = buf[i]
                e = jnp.exp(v)
                obuf[i] = jnp.where(e >= tau, v - logz, -70.0)
                return 0

            lax.fori_loop(0, _CH, row, 0)
            pltpu.sync_copy(
                obuf, o_hbm.at[b, pl.ds(ci * _CH, _CH), pl.ds(col0, _L)])
            return 0

        lax.fori_loop(0, nchunks, scan3_chunk, 0)
        return 0

    lax.fori_loop(0, (B * C // _L) // _NW, run_unit, 0)


def kernel(logits):
    B, V, C = logits.shape
    mesh = plsc.VectorSubcoreMesh(
        core_axis_name="c", subcore_axis_name="s", num_cores=2, num_subcores=16)
    f = pl.kernel(
        functools.partial(_sc_body),
        out_type=jax.ShapeDtypeStruct((B, V, C), jnp.float32),
        mesh=mesh,
        compiler_params=pltpu.CompilerParams(
            use_tc_tiling_on_sc=False, needs_layout_passes=False),
        scratch_types=[
            pltpu.VMEM((_CH, _L), jnp.float32),
            pltpu.VMEM((_CH, _L), jnp.float32),
            pltpu.VMEM((_HPAD, _L), jnp.float32),
            pltpu.VMEM((_NCAND + 8, _L), jnp.float32),
            pltpu.VMEM((_NCAND + 8, _L), jnp.int32),
        ],
    )
    return f(logits)
